# baseline (speedup 1.0000x reference)
import functools

import jax
import jax.numpy as jnp
from jax import lax
from jax.experimental import pallas as pl
from jax.experimental.pallas import tpu as pltpu
from jax.experimental.pallas import tpu_sc as plsc

_DIM = 1024
_SCALE = _DIM ** (-0.5)

_NC = 2
_NS = 16
_NW = _NC * _NS
_LANES = 16

_CHUNK = 32 * 1024


def _sc_scale_copy(total_elems):
    n_per_w = total_elems // _NW
    n_chunks = n_per_w // _CHUNK
    mesh = plsc.VectorSubcoreMesh(
        core_axis_name="c", subcore_axis_name="s",
        num_cores=_NC, num_subcores=_NS,
    )

    @functools.partial(
        pl.kernel,
        mesh=mesh,
        out_type=jax.ShapeDtypeStruct((total_elems,), jnp.float32),
        scratch_types=[
            pltpu.VMEM((_CHUNK,), jnp.float32),
            pltpu.VMEM((_CHUNK,), jnp.float32),
            pltpu.SemaphoreType.DMA,
            pltpu.SemaphoreType.DMA,
            pltpu.SemaphoreType.DMA,
            pltpu.SemaphoreType.DMA,
        ],
    )
    def k(emb_hbm, out_hbm, buf0, buf1, isem0, isem1, osem0, osem1):
        wid = lax.axis_index("s") * _NC + lax.axis_index("c")
        base = wid * n_per_w
        bufs = (buf0, buf1)
        isems = (isem0, isem1)
        osems = (osem0, osem1)

        in_h = [None] * n_chunks
        out_h = [None] * n_chunks
        in_h[0] = pltpu.async_copy(
            emb_hbm.at[pl.ds(base, _CHUNK)], buf0, isem0)
        for c in range(n_chunks):
            b = c % 2
            if c + 1 < n_chunks:
                if c >= 1:
                    out_h[c - 1].wait()
                nb = (c + 1) % 2
                in_h[c + 1] = pltpu.async_copy(
                    emb_hbm.at[pl.ds(base + (c + 1) * _CHUNK, _CHUNK)],
                    bufs[nb], isems[nb])
            in_h[c].wait()
            buf = bufs[b]

            @plsc.parallel_loop(0, _CHUNK, step=_LANES, unroll=8)
            def _(i):
                buf[pl.ds(i, _LANES)] = buf[pl.ds(i, _LANES)] * _SCALE

            out_h[c] = pltpu.async_copy(
                buf, out_hbm.at[pl.ds(base + c * _CHUNK, _CHUNK)], osems[b])
        if n_chunks >= 2:
            out_h[n_chunks - 2].wait()
        out_h[n_chunks - 1].wait()

    return k


def kernel(x, emb):
    seq_len = x.shape[1]
    total = seq_len * _DIM
    flat = emb.reshape(-1)[:total]
    out = _sc_scale_copy(total)(flat)
    return out.reshape(seq_len, _DIM)


# speedup vs baseline: 5.5153x; 5.5153x over previous
import jax
import jax.numpy as jnp
from jax.experimental import pallas as pl

_DIM = 1024
_SCALE = _DIM ** (-0.5)
_BLOCK_ROWS = 3584


def _scale_copy_body(emb_ref, o_ref):
    o_ref[...] = emb_ref[...] * _SCALE


def kernel(x, emb):
    seq_len = x.shape[1]
    emb_used = emb[:seq_len]
    grid = (pl.cdiv(seq_len, _BLOCK_ROWS),)
    return pl.pallas_call(
        _scale_copy_body,
        grid=grid,
        in_specs=[pl.BlockSpec((_BLOCK_ROWS, _DIM), lambda i: (i, 0))],
        out_specs=pl.BlockSpec((_BLOCK_ROWS, _DIM), lambda i: (i, 0)),
        out_shape=jax.ShapeDtypeStruct((seq_len, _DIM), emb.dtype),
    )(emb_used)
